# baseline (device time: 245859 ns/iter reference)
import os

import jax
import jax.numpy as jnp
import numpy as np
from jax import lax

_KMODE = os.environ.get("KMODE", "full")
from jax.experimental import pallas as pl
from jax.experimental.pallas import tpu as pltpu

N_DEV = 4
HPD = 8
HH = HPD // 2
DH = 128
SCALE = 0.08838834764831843
BLK = 64
QT = 256


def kernel(x, Wq, K_ext, V_ext, Wo):
    my = lax.axis_index("i")
    Sq, D = x.shape[1], x.shape[2]
    Skv = K_ext.shape[1]
    NH = K_ext.shape[2]
    HW = HH * DH

    xb = x[0].astype(jnp.bfloat16)
    wqa = (Wq[:, :HW] * SCALE).astype(jnp.bfloat16)
    wqb = (Wq[:, HW:] * SCALE).astype(jnp.bfloat16)
    woa = Wo[:HW, :].astype(jnp.bfloat16)
    wob = Wo[HW:, :].astype(jnp.bfloat16)
    k_any = K_ext.reshape(N_DEV, Skv, NH * DH)
    v_any = V_ext.reshape(N_DEV, Skv, NH * DH)
    qblk = np.arange(Sq)[:, None] // BLK
    kblk = np.arange(Skv)[None, :] // BLK
    bias = jnp.asarray(
        np.where(kblk <= qblk, 0.0, -1e9).astype(np.float32)
    ).astype(jnp.bfloat16)

    GW = HPD * DH

    def body(x_ref, wqa_ref, wqb_ref, woa_ref, wob_ref, k_any_ref, v_any_ref,
             bias_ref, out_ref,
             comm_wqa, comm_woa, comm_wqb, comm_wob,
             qa_ref, qb_ref, ctxa_ref, ctxb_ref, acc_ref,
             k32_ref, v32_ref, kbf_ref, vbf_ref,
             sa_q, ra_q, sa_o, ra_o, sb_q, rb_q, sb_o, rb_o,
             k_sem, v_sem):
        my_pos = lax.axis_index("i")
        left = lax.rem(my_pos + N_DEV - 1, N_DEV)
        right = lax.rem(my_pos + 1, N_DEV)

        barrier = pltpu.get_barrier_semaphore()
        for nbr in (left, right):
            pl.semaphore_signal(
                barrier, inc=1, device_id=(nbr,),
                device_id_type=pl.DeviceIdType.MESH,
            )
        pl.semaphore_wait(barrier, 2)

        def rc(src, dst, ssem, rsem, dev):
            return pltpu.make_async_remote_copy(
                src_ref=src, dst_ref=dst, send_sem=ssem, recv_sem=rsem,
                device_id=(dev,), device_id_type=pl.DeviceIdType.MESH,
            )

        def dma_step(si, slot):
            grs = lax.rem(my_pos - si + N_DEV, N_DEV)
            gls = lax.rem(my_pos + si, N_DEV)
            return [
                pltpu.make_async_copy(
                    k_any_ref.at[my_pos, :, pl.ds(grs * GW, HW)],
                    k32_ref.at[slot, :, pl.ds(0, HW)], k_sem.at[slot]),
                pltpu.make_async_copy(
                    k_any_ref.at[my_pos, :, pl.ds(gls * GW + HW, HW)],
                    k32_ref.at[slot, :, pl.ds(HW, HW)], k_sem.at[slot]),
                pltpu.make_async_copy(
                    v_any_ref.at[my_pos, :, pl.ds(grs * GW, HW)],
                    v32_ref.at[slot, :, pl.ds(0, HW)], v_sem.at[slot]),
                pltpu.make_async_copy(
                    v_any_ref.at[my_pos, :, pl.ds(gls * GW + HW, HW)],
                    v32_ref.at[slot, :, pl.ds(HW, HW)], v_sem.at[slot]),
            ]

        staging = not _KMODE.startswith("comm")
        pending = {}
        if staging:
            for si in (0, 1):
                pending[si] = dma_step(si, si)
                for d in pending[si]:
                    d.start()

        all_rdmas = []
        for s in range(N_DEV):
            wqa_s = wqa_ref if s == 0 else comm_wqa.at[s - 1]
            woa_s = woa_ref if s == 0 else comm_woa.at[s - 1]
            wqb_s = wqb_ref if s == 0 else comm_wqb.at[s - 1]
            wob_s = wob_ref if s == 0 else comm_wob.at[s - 1]
            if _KMODE == "comp":
                wqa_s, woa_s, wqb_s, wob_s = wqa_ref, woa_ref, wqb_ref, wob_ref
            if s < N_DEV - 1 and _KMODE != "comp":
                rdmas = [
                    rc(wqa_s, comm_wqa.at[s], sa_q.at[s], ra_q.at[s], right),
                    rc(woa_s, comm_woa.at[s], sa_o.at[s], ra_o.at[s], right),
                    rc(wqb_s, comm_wqb.at[s], sb_q.at[s], rb_q.at[s], left),
                    rc(wob_s, comm_wob.at[s], sb_o.at[s], rb_o.at[s], left),
                ]
                if _KMODE == "comm1":
                    rdmas = rdmas[:1] + rdmas[2:3]
                elif _KMODE == "commtiny":
                    rdmas = [
                        rc(wqa_s.at[0:16], comm_wqa.at[s, 0:16],
                           sa_q.at[s], ra_q.at[s], right),
                        rc(wqb_s.at[0:16], comm_wqb.at[s, 0:16],
                           sb_q.at[s], rb_q.at[s], left),
                    ]
                for r in rdmas:
                    r.start()

            gr = lax.rem(my_pos - s + N_DEV, N_DEV)
            gl = lax.rem(my_pos + s, N_DEV)
            if _KMODE.startswith("comm"):
                if s == 0:
                    out_ref[0] = jnp.zeros((Sq, D), jnp.bfloat16)
                out_ref[0, 0:1, 0:HW] += wqa_s[0:1, :] + wqb_s[0:1, :]
                out_ref[0, 1:2, :] += woa_s[0:1, :] + wob_s[0:1, :]
                if s < N_DEV - 1:
                    for r in rdmas:
                        r.wait()
                continue
            slot = s % 2
            for d in pending[s]:
                d.wait()
            kbf_ref[slot, :, :] = k32_ref[slot, :, :].astype(jnp.bfloat16)
            vbf_ref[slot, :, :] = v32_ref[slot, :, :].astype(jnp.bfloat16)
            if s + 2 < N_DEV:
                pending[s + 2] = dma_step(s + 2, slot)
                for d in pending[s + 2]:
                    d.start()

            qa_ref[...] = jnp.dot(
                x_ref[...], wqa_s[...], preferred_element_type=jnp.float32
            ).astype(jnp.bfloat16)
            qb_ref[...] = jnp.dot(
                x_ref[...], wqb_s[...], preferred_element_type=jnp.float32
            ).astype(jnp.bfloat16)

            def one_head(q_ref, ctx_ref, h, off, t):
                r0 = t * QT
                kend = (t + 1) * QT
                q = q_ref[pl.ds(r0, QT), pl.ds(h * DH, DH)]
                k = kbf_ref[slot, pl.ds(0, kend), pl.ds(off + h * DH, DH)]
                sc = lax.dot_general(
                    q, k, (((1,), (1,)), ((), ())),
                    preferred_element_type=jnp.float32,
                )
                w = jnp.exp(sc + bias_ref[pl.ds(r0, QT), pl.ds(0, kend)])
                denom = jnp.sum(w, axis=1, keepdims=True)
                ctx = jnp.dot(
                    w.astype(jnp.bfloat16),
                    vbf_ref[slot, pl.ds(0, kend), pl.ds(off + h * DH, DH)],
                    preferred_element_type=jnp.float32,
                )
                ctx_ref[pl.ds(r0, QT), pl.ds(h * DH, DH)] = (
                    ctx * (1.0 / denom)
                ).astype(jnp.bfloat16)

            for t in range(Sq // QT):
                def head_body(h, carry):
                    one_head(qa_ref, ctxa_ref, h, 0, t)
                    one_head(qb_ref, ctxb_ref, h, HW, t)
                    return carry

                lax.fori_loop(0, HH, head_body, 0)

            part = jnp.dot(
                ctxa_ref[...], woa_s[...], preferred_element_type=jnp.float32
            ) + jnp.dot(
                ctxb_ref[...], wob_s[...], preferred_element_type=jnp.float32
            )
            if s == 0:
                acc_ref[...] = part
            elif s < N_DEV - 1:
                acc_ref[...] += part
            else:
                out_ref[0] = (acc_ref[...] + part).astype(jnp.bfloat16)

            if s < N_DEV - 1 and _KMODE != "comp":
                for r in rdmas:
                    r.wait_recv()
                all_rdmas.extend(rdmas)

        for r in all_rdmas:
            r.wait_send()

    nh = N_DEV - 1
    return pl.pallas_call(
        body,
        out_shape=jax.ShapeDtypeStruct((1, Sq, D), jnp.bfloat16),
        in_specs=[pl.BlockSpec(memory_space=pltpu.VMEM)] * 5
        + [pl.BlockSpec(memory_space=pl.ANY)] * 2
        + [pl.BlockSpec(memory_space=pltpu.VMEM)],
        out_specs=pl.BlockSpec(memory_space=pltpu.VMEM),
        scratch_shapes=[
            pltpu.VMEM((nh, D, HW), jnp.bfloat16),
            pltpu.VMEM((nh, HW, D), jnp.bfloat16),
            pltpu.VMEM((nh, D, HW), jnp.bfloat16),
            pltpu.VMEM((nh, HW, D), jnp.bfloat16),
            pltpu.VMEM((Sq, HW), jnp.bfloat16),
            pltpu.VMEM((Sq, HW), jnp.bfloat16),
            pltpu.VMEM((Sq, HW), jnp.bfloat16),
            pltpu.VMEM((Sq, HW), jnp.bfloat16),
            pltpu.VMEM((Sq, D), jnp.float32),
            pltpu.VMEM((2, Skv, GW), jnp.float32),
            pltpu.VMEM((2, Skv, GW), jnp.float32),
            pltpu.VMEM((2, Skv, GW), jnp.bfloat16),
            pltpu.VMEM((2, Skv, GW), jnp.bfloat16),
        ] + [pltpu.SemaphoreType.DMA((nh,))] * 8
        + [pltpu.SemaphoreType.DMA((2,))] * 2,
        compiler_params=pltpu.CompilerParams(
            collective_id=0, vmem_limit_bytes=100 * 1024 * 1024
        ),
    )(xb, wqa, wqb, woa, wob, k_any, v_any, bias)


# device time: 134752 ns/iter; 1.8245x vs baseline; 1.8245x over previous
import os

import jax
import jax.numpy as jnp
import numpy as np
from jax import lax

_KMODE = os.environ.get("KMODE", "full")
from jax.experimental import pallas as pl
from jax.experimental.pallas import tpu as pltpu

N_DEV = 4
HPD = 8
HH = HPD // 2
DH = 128
SCALE = 0.08838834764831843
BLK = 64
QT = 512


def kernel(x, Wq, K_ext, V_ext, Wo):
    my = lax.axis_index("i")
    Sq, D = x.shape[1], x.shape[2]
    Skv = K_ext.shape[1]
    NH = K_ext.shape[2]
    HW = HH * DH

    xb = x[0].astype(jnp.bfloat16)
    wqa = (Wq[:, :HW] * SCALE).astype(jnp.bfloat16)
    wqb = (Wq[:, HW:] * SCALE).astype(jnp.bfloat16)
    woa = Wo[:HW, :].astype(jnp.bfloat16)
    wob = Wo[HW:, :].astype(jnp.bfloat16)
    k_my = lax.dynamic_index_in_dim(K_ext, my, 0, keepdims=False).reshape(
        Skv, NH * DH
    ).astype(jnp.bfloat16)
    v_t = lax.dynamic_index_in_dim(V_ext, my, 0, keepdims=False).reshape(
        Skv, NH * DH
    ).astype(jnp.bfloat16)
    qblk = np.arange(Sq)[:, None] // BLK
    kblk = np.arange(Skv)[None, :] // BLK
    bias = jnp.asarray(
        np.where(kblk <= qblk, 0.0, -1e9).astype(np.float32)
    ).astype(jnp.bfloat16)

    GW = HPD * DH

    def body(x_ref, wqa_ref, wqb_ref, woa_ref, wob_ref, k_ref, v_ref,
             bias_ref, out_ref,
             comm_wqa, comm_woa, comm_wqb, comm_wob,
             qa_ref, qb_ref, ctxa_ref, ctxb_ref, acc_ref,
             sa_q, ra_q, sa_o, ra_o, sb_q, rb_q, sb_o, rb_o):
        my_pos = lax.axis_index("i")
        left = lax.rem(my_pos + N_DEV - 1, N_DEV)
        right = lax.rem(my_pos + 1, N_DEV)

        barrier = pltpu.get_barrier_semaphore()
        for nbr in (left, right):
            pl.semaphore_signal(
                barrier, inc=1, device_id=(nbr,),
                device_id_type=pl.DeviceIdType.MESH,
            )
        pl.semaphore_wait(barrier, 2)

        def rc(src, dst, ssem, rsem, dev):
            return pltpu.make_async_remote_copy(
                src_ref=src, dst_ref=dst, send_sem=ssem, recv_sem=rsem,
                device_id=(dev,), device_id_type=pl.DeviceIdType.MESH,
            )


        all_rdmas = []
        for s in range(N_DEV):
            wqa_s = wqa_ref if s == 0 else comm_wqa.at[s - 1]
            woa_s = woa_ref if s == 0 else comm_woa.at[s - 1]
            wqb_s = wqb_ref if s == 0 else comm_wqb.at[s - 1]
            wob_s = wob_ref if s == 0 else comm_wob.at[s - 1]
            if _KMODE == "comp":
                wqa_s, woa_s, wqb_s, wob_s = wqa_ref, woa_ref, wqb_ref, wob_ref
            if s < N_DEV - 1 and _KMODE != "comp":
                rdmas = [
                    rc(wqa_s, comm_wqa.at[s], sa_q.at[s], ra_q.at[s], right),
                    rc(woa_s, comm_woa.at[s], sa_o.at[s], ra_o.at[s], right),
                    rc(wqb_s, comm_wqb.at[s], sb_q.at[s], rb_q.at[s], left),
                    rc(wob_s, comm_wob.at[s], sb_o.at[s], rb_o.at[s], left),
                ]
                if _KMODE == "comm1":
                    rdmas = rdmas[:1] + rdmas[2:3]
                elif _KMODE == "commtiny":
                    rdmas = [
                        rc(wqa_s.at[0:16], comm_wqa.at[s, 0:16],
                           sa_q.at[s], ra_q.at[s], right),
                        rc(wqb_s.at[0:16], comm_wqb.at[s, 0:16],
                           sb_q.at[s], rb_q.at[s], left),
                    ]
                for r in rdmas:
                    r.start()

            gr = lax.rem(my_pos - s + N_DEV, N_DEV)
            gl = lax.rem(my_pos + s, N_DEV)
            if _KMODE.startswith("comm"):
                if s == 0:
                    out_ref[0] = jnp.zeros((Sq, D), jnp.bfloat16)
                out_ref[0, 0:1, 0:HW] += wqa_s[0:1, :] + wqb_s[0:1, :]
                out_ref[0, 1:2, :] += woa_s[0:1, :] + wob_s[0:1, :]
                if s < N_DEV - 1:
                    for r in rdmas:
                        r.wait()
                continue
            qa_ref[...] = jnp.dot(
                x_ref[...], wqa_s[...], preferred_element_type=jnp.float32
            ).astype(jnp.bfloat16)
            qb_ref[...] = jnp.dot(
                x_ref[...], wqb_s[...], preferred_element_type=jnp.float32
            ).astype(jnp.bfloat16)

            def one_head(q_ref, ctx_ref, h, head, t):
                r0 = t * QT
                kend = (t + 1) * QT
                q = q_ref[pl.ds(r0, QT), pl.ds(h * DH, DH)]
                k = k_ref[pl.ds(0, kend), pl.ds(head * DH, DH)]
                sc = lax.dot_general(
                    q, k, (((1,), (1,)), ((), ())),
                    preferred_element_type=jnp.float32,
                )
                w = jnp.exp(sc + bias_ref[pl.ds(r0, QT), pl.ds(0, kend)])
                denom = jnp.sum(w, axis=1, keepdims=True)
                ctx = jnp.dot(
                    w.astype(jnp.bfloat16),
                    v_ref[pl.ds(0, kend), pl.ds(head * DH, DH)],
                    preferred_element_type=jnp.float32,
                )
                ctx_ref[pl.ds(r0, QT), pl.ds(h * DH, DH)] = (
                    ctx * (1.0 / denom)
                ).astype(jnp.bfloat16)

            for t in range(Sq // QT):
                def head_body(h, carry):
                    one_head(qa_ref, ctxa_ref, h, gr * HPD + h, t)
                    one_head(qb_ref, ctxb_ref, h, gl * HPD + HH + h, t)
                    return carry

                lax.fori_loop(0, HH, head_body, 0)

            part = jnp.dot(
                ctxa_ref[...], woa_s[...], preferred_element_type=jnp.float32
            ) + jnp.dot(
                ctxb_ref[...], wob_s[...], preferred_element_type=jnp.float32
            )
            if s == 0:
                acc_ref[...] = part
            elif s < N_DEV - 1:
                acc_ref[...] += part
            else:
                out_ref[0] = (acc_ref[...] + part).astype(jnp.bfloat16)

            if s < N_DEV - 1 and _KMODE != "comp":
                for r in rdmas:
                    r.wait_recv()
                all_rdmas.extend(rdmas)

        for r in all_rdmas:
            r.wait_send()

    nh = N_DEV - 1
    return pl.pallas_call(
        body,
        out_shape=jax.ShapeDtypeStruct((1, Sq, D), jnp.bfloat16),
        in_specs=[pl.BlockSpec(memory_space=pltpu.VMEM)] * 8,
        out_specs=pl.BlockSpec(memory_space=pltpu.VMEM),
        scratch_shapes=[
            pltpu.VMEM((nh, D, HW), jnp.bfloat16),
            pltpu.VMEM((nh, HW, D), jnp.bfloat16),
            pltpu.VMEM((nh, D, HW), jnp.bfloat16),
            pltpu.VMEM((nh, HW, D), jnp.bfloat16),
            pltpu.VMEM((Sq, HW), jnp.bfloat16),
            pltpu.VMEM((Sq, HW), jnp.bfloat16),
            pltpu.VMEM((Sq, HW), jnp.bfloat16),
            pltpu.VMEM((Sq, HW), jnp.bfloat16),
            pltpu.VMEM((Sq, D), jnp.float32),
        ] + [pltpu.SemaphoreType.DMA((nh,))] * 8,
        compiler_params=pltpu.CompilerParams(
            collective_id=0, vmem_limit_bytes=100 * 1024 * 1024
        ),
    )(xb, wqa, wqb, woa, wob, k_my, v_t, bias)


# device time: 134225 ns/iter; 1.8317x vs baseline; 1.0039x over previous
import os

import jax
import jax.numpy as jnp
import numpy as np
from jax import lax

_KMODE = os.environ.get("KMODE", "full")
from jax.experimental import pallas as pl
from jax.experimental.pallas import tpu as pltpu

N_DEV = 4
HPD = 8
HH = HPD // 2
DH = 128
SCALE = 0.08838834764831843
BLK = 64
QT = 512


def kernel(x, Wq, K_ext, V_ext, Wo):
    my = lax.axis_index("i")
    Sq, D = x.shape[1], x.shape[2]
    Skv = K_ext.shape[1]
    NH = K_ext.shape[2]
    HW = HH * DH

    xb = x[0].astype(jnp.bfloat16)
    wqa = (Wq[:, :HW] * SCALE).astype(jnp.bfloat16)
    wqb = (Wq[:, HW:] * SCALE).astype(jnp.bfloat16)
    woa = Wo[:HW, :].astype(jnp.bfloat16)
    wob = Wo[HW:, :].astype(jnp.bfloat16)
    k_my = lax.dynamic_index_in_dim(K_ext, my, 0, keepdims=False).reshape(
        Skv, NH * DH
    ).astype(jnp.bfloat16)
    v_t = lax.dynamic_index_in_dim(V_ext, my, 0, keepdims=False).reshape(
        Skv, NH * DH
    ).astype(jnp.bfloat16)
    qblk = np.arange(Sq)[:, None] // BLK
    kblk = np.arange(Skv)[None, :] // BLK
    bias = jnp.asarray(
        np.where(kblk <= qblk, 0.0, -1e9).astype(np.float32)
    ).astype(jnp.bfloat16)

    def body(x_ref, wqa_ref, wqb_ref, woa_ref, wob_ref, k_ref, v_ref,
             bias_ref, out_ref,
             comm_wqa, comm_woa, comm_wqb, comm_wob,
             qa_ref, qb_ref, ctxa_ref, ctxb_ref, acc_ref,
             sa_q, ra_q, sa_o, ra_o, sb_q, rb_q, sb_o, rb_o):
        my_pos = lax.axis_index("i")
        left = lax.rem(my_pos + N_DEV - 1, N_DEV)
        right = lax.rem(my_pos + 1, N_DEV)

        barrier = pltpu.get_barrier_semaphore()
        for nbr in (left, right):
            pl.semaphore_signal(
                barrier, inc=1, device_id=(nbr,),
                device_id_type=pl.DeviceIdType.MESH,
            )
        pl.semaphore_wait(barrier, 2)

        def rc(src, dst, ssem, rsem, dev):
            return pltpu.make_async_remote_copy(
                src_ref=src, dst_ref=dst, send_sem=ssem, recv_sem=rsem,
                device_id=(dev,), device_id_type=pl.DeviceIdType.MESH,
            )


        all_rdmas = []
        for s in range(N_DEV):
            wqa_s = wqa_ref if s == 0 else comm_wqa.at[s - 1]
            woa_s = woa_ref if s == 0 else comm_woa.at[s - 1]
            wqb_s = wqb_ref if s == 0 else comm_wqb.at[s - 1]
            wob_s = wob_ref if s == 0 else comm_wob.at[s - 1]
            if _KMODE == "comp":
                wqa_s, woa_s, wqb_s, wob_s = wqa_ref, woa_ref, wqb_ref, wob_ref
            if s < N_DEV - 1 and _KMODE != "comp":
                rdmas = [
                    rc(wqa_s, comm_wqa.at[s], sa_q.at[s], ra_q.at[s], right),
                    rc(woa_s, comm_woa.at[s], sa_o.at[s], ra_o.at[s], right),
                    rc(wqb_s, comm_wqb.at[s], sb_q.at[s], rb_q.at[s], left),
                    rc(wob_s, comm_wob.at[s], sb_o.at[s], rb_o.at[s], left),
                ]
                if _KMODE == "comm1":
                    rdmas = rdmas[:1] + rdmas[2:3]
                elif _KMODE == "commtiny":
                    rdmas = [
                        rc(wqa_s.at[0:16], comm_wqa.at[s, 0:16],
                           sa_q.at[s], ra_q.at[s], right),
                        rc(wqb_s.at[0:16], comm_wqb.at[s, 0:16],
                           sb_q.at[s], rb_q.at[s], left),
                    ]
                for r in rdmas:
                    r.start()

            gr = lax.rem(my_pos - s + N_DEV, N_DEV)
            gl = lax.rem(my_pos + s, N_DEV)
            if _KMODE.startswith("comm"):
                if s == 0:
                    out_ref[0] = jnp.zeros((Sq, D), jnp.bfloat16)
                out_ref[0, 0:1, 0:HW] += wqa_s[0:1, :] + wqb_s[0:1, :]
                out_ref[0, 1:2, :] += woa_s[0:1, :] + wob_s[0:1, :]
                if s < N_DEV - 1:
                    for r in rdmas:
                        r.wait()
                continue
            qa_ref[...] = jnp.dot(
                x_ref[...], wqa_s[...], preferred_element_type=jnp.float32
            ).astype(jnp.bfloat16)
            qb_ref[...] = jnp.dot(
                x_ref[...], wqb_s[...], preferred_element_type=jnp.float32
            ).astype(jnp.bfloat16)

            def one_head(q_ref, ctx_ref, h, head, t):
                r0 = t * QT
                kend = (t + 1) * QT
                q = q_ref[pl.ds(r0, QT), pl.ds(h * DH, DH)]
                k = k_ref[pl.ds(0, kend), pl.ds(head * DH, DH)]
                sc = lax.dot_general(
                    q, k, (((1,), (1,)), ((), ())),
                    preferred_element_type=jnp.float32,
                )
                w = jnp.exp(sc + bias_ref[pl.ds(r0, QT), pl.ds(0, kend)])
                denom = jnp.sum(w, axis=1, keepdims=True)
                ctx = jnp.dot(
                    w.astype(jnp.bfloat16),
                    v_ref[pl.ds(0, kend), pl.ds(head * DH, DH)],
                    preferred_element_type=jnp.float32,
                )
                ctx_ref[pl.ds(r0, QT), pl.ds(h * DH, DH)] = (
                    ctx * (1.0 / denom)
                ).astype(jnp.bfloat16)

            for t in range(Sq // QT):
                def head_body(h, carry):
                    one_head(qa_ref, ctxa_ref, h, gr * HPD + h, t)
                    one_head(qb_ref, ctxb_ref, h, gl * HPD + HH + h, t)
                    return carry

                lax.fori_loop(0, HH, head_body, 0)

            part = jnp.dot(
                ctxa_ref[...], woa_s[...], preferred_element_type=jnp.float32
            ) + jnp.dot(
                ctxb_ref[...], wob_s[...], preferred_element_type=jnp.float32
            )
            if s == 0:
                acc_ref[...] = part
            elif s < N_DEV - 1:
                acc_ref[...] += part
            else:
                out_ref[0] = (acc_ref[...] + part).astype(jnp.bfloat16)

            if s < N_DEV - 1 and _KMODE != "comp":
                for r in rdmas:
                    r.wait_recv()
                all_rdmas.extend(rdmas)

        for r in all_rdmas:
            r.wait_send()

    nh = N_DEV - 1
    return pl.pallas_call(
        body,
        out_shape=jax.ShapeDtypeStruct((1, Sq, D), jnp.bfloat16),
        in_specs=[pl.BlockSpec(memory_space=pltpu.VMEM)] * 8,
        out_specs=pl.BlockSpec(memory_space=pltpu.VMEM),
        scratch_shapes=[
            pltpu.VMEM((nh, D, HW), jnp.bfloat16),
            pltpu.VMEM((nh, HW, D), jnp.bfloat16),
            pltpu.VMEM((nh, D, HW), jnp.bfloat16),
            pltpu.VMEM((nh, HW, D), jnp.bfloat16),
            pltpu.VMEM((Sq, HW), jnp.bfloat16),
            pltpu.VMEM((Sq, HW), jnp.bfloat16),
            pltpu.VMEM((Sq, HW), jnp.bfloat16),
            pltpu.VMEM((Sq, HW), jnp.bfloat16),
            pltpu.VMEM((Sq, D), jnp.float32),
        ] + [pltpu.SemaphoreType.DMA((nh,))] * 8,
        compiler_params=pltpu.CompilerParams(
            collective_id=0, vmem_limit_bytes=100 * 1024 * 1024
        ),
    )(xb, wqa, wqb, woa, wob, k_my, v_t, bias)
